# speedup vs baseline: 13.0306x; 13.0306x over previous
import functools

import jax
import jax.numpy as jnp
from jax import lax
from jax.experimental import pallas as pl
from jax.experimental.pallas import tpu as pltpu
from jax.experimental.pallas import tpu_sc as plsc

N_NODES = 10000
N_EDGES = 320000
D = 128

NC = 2
NS = 16
NW = NC * NS

EPW = N_EDGES // NW
CHUNK = 80
NCHUNK = EPW // CHUNK

NP = 10240
RPS = NP // NS

ROW_BLK = 1000
N_BLKS = N_NODES // ROW_BLK

_sc_mesh = plsc.VectorSubcoreMesh(core_axis_name="c", subcore_axis_name="s")


@functools.partial(
    pl.kernel,
    out_type=jax.ShapeDtypeStruct((NC, NP), jnp.float32),
    mesh=_sc_mesh,
    scratch_types=[
        pltpu.VMEM((CHUNK,), jnp.int32),
        pltpu.VMEM((CHUNK,), jnp.float32),
        pltpu.VMEM_SHARED((NP,), jnp.float32),
    ],
)
def _deg_kernel(dst_hbm, ones_hbm, zeros_hbm, deg_out, idx_v, ones_v, deg_sh):
    cid = lax.axis_index("c")
    sid = lax.axis_index("s")
    wid = sid * NC + cid
    pltpu.sync_copy(zeros_hbm, deg_sh.at[pl.ds(sid * RPS, RPS)])
    pltpu.sync_copy(ones_hbm, ones_v)
    plsc.subcore_barrier()

    def body(i, carry):
        base = wid * EPW + i * CHUNK
        pltpu.sync_copy(dst_hbm.at[pl.ds(base, CHUNK)], idx_v)
        pltpu.sync_copy(ones_v, deg_sh.at[idx_v], add=True)
        return carry

    lax.fori_loop(0, NCHUNK, body, 0)
    plsc.subcore_barrier()
    pltpu.sync_copy(deg_sh.at[pl.ds(sid * RPS, RPS)],
                    deg_out.at[cid, pl.ds(sid * RPS, RPS)])


@functools.partial(
    pl.kernel,
    out_type=jax.ShapeDtypeStruct((NC, NP, D), jnp.float32),
    mesh=_sc_mesh,
    scratch_types=[
        pltpu.VMEM((CHUNK,), jnp.int32),
        pltpu.VMEM((CHUNK,), jnp.int32),
        pltpu.VMEM((CHUNK, D), jnp.float32),
        pltpu.VMEM_SHARED((NP, D), jnp.float32),
        pltpu.SemaphoreType.DMA,
    ],
)
def _msg_kernel(y_hbm, src_hbm, dst_hbm, zrows_hbm, acc_out,
                src_v, dst_v, rows_v, acc_sh, sem):
    cid = lax.axis_index("c")
    sid = lax.axis_index("s")
    wid = sid * NC + cid
    pltpu.sync_copy(zrows_hbm, acc_sh.at[pl.ds(sid * RPS, RPS)])
    plsc.subcore_barrier()

    def body(i, carry):
        base = wid * EPW + i * CHUNK
        pltpu.sync_copy(src_hbm.at[pl.ds(base, CHUNK)], src_v)
        pltpu.sync_copy(dst_hbm.at[pl.ds(base, CHUNK)], dst_v)
        pltpu.async_copy(y_hbm.at[src_v], rows_v, sem).wait()
        pltpu.sync_copy(rows_v, acc_sh.at[dst_v], add=True)
        return carry

    lax.fori_loop(0, NCHUNK, body, 0)
    plsc.subcore_barrier()
    pltpu.sync_copy(acc_sh.at[pl.ds(sid * RPS, RPS)],
                    acc_out.at[cid, pl.ds(sid * RPS, RPS)])


def _dinv(d0, d1):
    return lax.rsqrt(d0 + d1 + 1.0)


def _l1_body(x_ref, w1_ref, d0_ref, d1_ref, y_ref):
    dinv = _dinv(d0_ref[...], d1_ref[...])
    y_ref[...] = jnp.dot(x_ref[...], w1_ref[...],
                         preferred_element_type=jnp.float32) * dinv


def _l2_body(a0_ref, a1_ref, y1_ref, d0_ref, d1_ref, b1_ref, w2_ref, y2_ref):
    dinv = _dinv(d0_ref[...], d1_ref[...])
    h = dinv * (a0_ref[...] + a1_ref[...] + y1_ref[...]) + b1_ref[...]
    h = jnp.maximum(h, 0.0)
    y2_ref[...] = jnp.dot(h, w2_ref[...],
                          preferred_element_type=jnp.float32) * dinv


def _out_body(a0_ref, a1_ref, y2_ref, d0_ref, d1_ref, b2_ref, o_ref):
    dinv = _dinv(d0_ref[...], d1_ref[...])
    logits = dinv * (a0_ref[...] + a1_ref[...] + y2_ref[...]) + b2_ref[...]
    m = jnp.max(logits, axis=1, keepdims=True)
    lse = jnp.log(jnp.sum(jnp.exp(logits - m), axis=1, keepdims=True)) + m
    o_ref[...] = logits - lse


def _row_spec():
    return pl.BlockSpec((ROW_BLK, D), lambda i: (i, 0))


def _deg_spec():
    return pl.BlockSpec((ROW_BLK, 1), lambda i: (i, 0))


def _full_spec():
    return pl.BlockSpec((D, D), lambda i: (0, 0))


def _bias_spec():
    return pl.BlockSpec((1, D), lambda i: (0, 0))


def kernel(x, edge_index, W1, b1, W2, b2):
    src = edge_index[0].astype(jnp.int32)
    dst = edge_index[1].astype(jnp.int32)

    ones_c = jnp.ones((CHUNK,), jnp.float32)
    zeros_1d = jnp.zeros((RPS,), jnp.float32)
    zeros_2d = jnp.zeros((RPS, D), jnp.float32)

    degp = _deg_kernel(dst, ones_c, zeros_1d)
    d0 = degp[0, :N_NODES].reshape(N_NODES, 1)
    d1 = degp[1, :N_NODES].reshape(N_NODES, 1)

    y1 = pl.pallas_call(
        _l1_body,
        grid=(N_BLKS,),
        in_specs=[_row_spec(), _full_spec(), _deg_spec(), _deg_spec()],
        out_specs=_row_spec(),
        out_shape=jax.ShapeDtypeStruct((N_NODES, D), jnp.float32),
    )(x, W1, d0, d1)

    accp1 = _msg_kernel(y1, src, dst, zeros_2d)

    y2 = pl.pallas_call(
        _l2_body,
        grid=(N_BLKS,),
        in_specs=[_row_spec(), _row_spec(), _row_spec(), _deg_spec(),
                  _deg_spec(), _bias_spec(), _full_spec()],
        out_specs=_row_spec(),
        out_shape=jax.ShapeDtypeStruct((N_NODES, D), jnp.float32),
    )(accp1[0, :N_NODES], accp1[1, :N_NODES], y1, d0, d1,
      b1.reshape(1, D), W2)

    accp2 = _msg_kernel(y2, src, dst, zeros_2d)

    out = pl.pallas_call(
        _out_body,
        grid=(N_BLKS,),
        in_specs=[_row_spec(), _row_spec(), _row_spec(), _deg_spec(),
                  _deg_spec(), _bias_spec()],
        out_specs=_row_spec(),
        out_shape=jax.ShapeDtypeStruct((N_NODES, D), jnp.float32),
    )(accp2[0, :N_NODES], accp2[1, :N_NODES], y2, d0, d1, b2.reshape(1, D))

    return out


# speedup vs baseline: 25.5774x; 1.9629x over previous
import functools

import jax
import jax.numpy as jnp
from jax import lax
from jax.experimental import pallas as pl
from jax.experimental.pallas import tpu as pltpu
from jax.experimental.pallas import tpu_sc as plsc

N_NODES = 10000
N_EDGES = 320000
D = 128

NC = 2
NS = 16
NW = NC * NS

EPW = N_EDGES // NW
CHUNK = 80
NCHUNK = EPW // CHUNK
NPAIR = NCHUNK // 2

NP = 10240
RPS = NP // NS

ROW_BLK = 1000
N_BLKS = N_NODES // ROW_BLK

_sc_mesh = plsc.VectorSubcoreMesh(core_axis_name="c", subcore_axis_name="s")


@functools.partial(
    pl.kernel,
    out_type=jax.ShapeDtypeStruct((NC, NP), jnp.float32),
    mesh=_sc_mesh,
    scratch_types=[
        pltpu.VMEM((CHUNK,), jnp.int32),
        pltpu.VMEM((CHUNK,), jnp.float32),
        pltpu.VMEM_SHARED((NP,), jnp.float32),
    ],
)
def _deg_kernel(dst_hbm, ones_hbm, zeros_hbm, deg_out, idx_v, ones_v, deg_sh):
    cid = lax.axis_index("c")
    sid = lax.axis_index("s")
    wid = sid * NC + cid
    pltpu.sync_copy(zeros_hbm, deg_sh.at[pl.ds(sid * RPS, RPS)])
    pltpu.sync_copy(ones_hbm, ones_v)
    plsc.subcore_barrier()

    def body(i, carry):
        base = wid * EPW + i * CHUNK
        pltpu.sync_copy(dst_hbm.at[pl.ds(base, CHUNK)], idx_v)
        pltpu.sync_copy(ones_v, deg_sh.at[idx_v], add=True)
        return carry

    lax.fori_loop(0, NCHUNK, body, 0)
    plsc.subcore_barrier()
    pltpu.sync_copy(deg_sh.at[pl.ds(sid * RPS, RPS)],
                    deg_out.at[cid, pl.ds(sid * RPS, RPS)])


@functools.partial(
    pl.kernel,
    out_type=jax.ShapeDtypeStruct((NC, NP, D), jnp.float32),
    mesh=_sc_mesh,
    scratch_types=[
        pltpu.VMEM((NCHUNK, 1, CHUNK), jnp.int32),
        pltpu.VMEM((CHUNK,), jnp.int32),
        pltpu.VMEM((CHUNK,), jnp.int32),
        pltpu.VMEM((CHUNK, D), jnp.float32),
        pltpu.VMEM((CHUNK, D), jnp.float32),
        pltpu.VMEM_SHARED((NP, D), jnp.float32),
        pltpu.SemaphoreType.DMA,
        pltpu.SemaphoreType.DMA,
    ],
)
def _msg_kernel(y_hbm, src_hbm, dst_hbm, zrows_hbm, acc_out,
                src_v, dst0, dst1, rows0, rows1, acc_sh, gsem0, gsem1):
    cid = lax.axis_index("c")
    sid = lax.axis_index("s")
    wid = sid * NC + cid
    pltpu.sync_copy(zrows_hbm, acc_sh.at[pl.ds(sid * RPS, RPS)])
    pltpu.sync_copy(src_hbm.at[wid], src_v)
    plsc.subcore_barrier()

    def fetch(j, dst_b, rows_b, sem):
        pltpu.async_copy(dst_hbm.at[wid, j, 0], dst_b, sem)
        pltpu.async_copy(y_hbm.at[src_v.at[j, 0]], rows_b, sem)

    def drain_and_add(j, dst_b, rows_b, sem):
        pltpu.make_async_copy(dst_hbm.at[wid, j, 0], dst_b, sem).wait()
        pltpu.make_async_copy(y_hbm.at[src_v.at[j, 0]], rows_b, sem).wait()
        pltpu.sync_copy(rows_b, acc_sh.at[dst_b], add=True)

    fetch(0, dst0, rows0, gsem0)

    def body(i, carry):
        j0 = 2 * i
        j1 = j0 + 1
        fetch(j1, dst1, rows1, gsem1)
        drain_and_add(j0, dst0, rows0, gsem0)
        fetch(j0 + 2, dst0, rows0, gsem0)
        drain_and_add(j1, dst1, rows1, gsem1)
        return carry

    lax.fori_loop(0, NPAIR, body, 0)
    drain_and_add(NCHUNK - 1, dst0, rows0, gsem0)
    plsc.subcore_barrier()
    pltpu.sync_copy(acc_sh.at[pl.ds(sid * RPS, RPS)],
                    acc_out.at[cid, pl.ds(sid * RPS, RPS)])


def _dinv(d0, d1):
    return lax.rsqrt(d0 + d1 + 1.0)


def _l1_body(x_ref, w1_ref, d0_ref, d1_ref, y_ref):
    dinv = _dinv(d0_ref[...], d1_ref[...])
    y_ref[...] = jnp.dot(x_ref[...], w1_ref[...],
                         preferred_element_type=jnp.float32) * dinv


def _l2_body(a0_ref, a1_ref, y1_ref, d0_ref, d1_ref, b1_ref, w2_ref, y2_ref):
    dinv = _dinv(d0_ref[...], d1_ref[...])
    h = dinv * (a0_ref[...] + a1_ref[...] + y1_ref[...]) + b1_ref[...]
    h = jnp.maximum(h, 0.0)
    y2_ref[...] = jnp.dot(h, w2_ref[...],
                          preferred_element_type=jnp.float32) * dinv


def _out_body(a0_ref, a1_ref, y2_ref, d0_ref, d1_ref, b2_ref, o_ref):
    dinv = _dinv(d0_ref[...], d1_ref[...])
    logits = dinv * (a0_ref[...] + a1_ref[...] + y2_ref[...]) + b2_ref[...]
    m = jnp.max(logits, axis=1, keepdims=True)
    lse = jnp.log(jnp.sum(jnp.exp(logits - m), axis=1, keepdims=True)) + m
    o_ref[...] = logits - lse


def _row_spec():
    return pl.BlockSpec((ROW_BLK, D), lambda i: (i, 0))


def _deg_spec():
    return pl.BlockSpec((ROW_BLK, 1), lambda i: (i, 0))


def _full_spec():
    return pl.BlockSpec((D, D), lambda i: (0, 0))


def _bias_spec():
    return pl.BlockSpec((1, D), lambda i: (0, 0))


def kernel(x, edge_index, W1, b1, W2, b2):
    src = edge_index[0].astype(jnp.int32)
    dst = edge_index[1].astype(jnp.int32)
    src4 = src.reshape(NW, NCHUNK, 1, CHUNK)
    dst4 = dst.reshape(NW, NCHUNK, 1, CHUNK)

    ones_c = jnp.ones((CHUNK,), jnp.float32)
    zeros_1d = jnp.zeros((RPS,), jnp.float32)
    zeros_2d = jnp.zeros((RPS, D), jnp.float32)

    degp = _deg_kernel(dst, ones_c, zeros_1d)
    d0 = degp[0, :N_NODES].reshape(N_NODES, 1)
    d1 = degp[1, :N_NODES].reshape(N_NODES, 1)

    y1 = pl.pallas_call(
        _l1_body,
        grid=(N_BLKS,),
        in_specs=[_row_spec(), _full_spec(), _deg_spec(), _deg_spec()],
        out_specs=_row_spec(),
        out_shape=jax.ShapeDtypeStruct((N_NODES, D), jnp.float32),
    )(x, W1, d0, d1)

    accp1 = _msg_kernel(y1, src4, dst4, zeros_2d)

    y2 = pl.pallas_call(
        _l2_body,
        grid=(N_BLKS,),
        in_specs=[_row_spec(), _row_spec(), _row_spec(), _deg_spec(),
                  _deg_spec(), _bias_spec(), _full_spec()],
        out_specs=_row_spec(),
        out_shape=jax.ShapeDtypeStruct((N_NODES, D), jnp.float32),
    )(accp1[0, :N_NODES], accp1[1, :N_NODES], y1, d0, d1,
      b1.reshape(1, D), W2)

    accp2 = _msg_kernel(y2, src4, dst4, zeros_2d)

    out = pl.pallas_call(
        _out_body,
        grid=(N_BLKS,),
        in_specs=[_row_spec(), _row_spec(), _row_spec(), _deg_spec(),
                  _deg_spec(), _bias_spec()],
        out_specs=_row_spec(),
        out_shape=jax.ShapeDtypeStruct((N_NODES, D), jnp.float32),
    )(accp2[0, :N_NODES], accp2[1, :N_NODES], y2, d0, d1, b2.reshape(1, D))

    return out
